# baseline (speedup 1.0000x reference)
import jax
import jax.numpy as jnp
from jax import lax
from jax.experimental import pallas as pl

N = 512
K = 32
K0 = 64
Q = 16
BT = 4096


def _builder(in_idx_ref, w_hid_ref, wp_ref, out_idx_ref, w_out_ref,
             ats_ref, ust_ref, gt_ref):
    iota_d = lax.broadcasted_iota(jnp.int32, (N, N), 0)
    iota_s = lax.broadcasted_iota(jnp.int32, (N, N), 1)
    at = jnp.zeros((N, N), jnp.float32)
    for k in range(K):
        idx_k = in_idx_ref[:, k:k + 1]
        at += w_hid_ref[:, k:k + 1] * (idx_k == iota_s).astype(jnp.float32)
    ats_ref[...] = at * (iota_s < iota_d).astype(jnp.float32)

    wp = wp_ref[...]
    iota_dc = lax.broadcasted_iota(jnp.int32, (N, K0), 0)
    ust = jnp.zeros((N, K0), jnp.float32)
    for k in range(K):
        idx_k = in_idx_ref[:, k:k + 1]
        ust += w_hid_ref[:, k:k + 1] * (idx_k == wp).astype(jnp.float32)
    ust_ref[...] = ust * (wp >= iota_dc).astype(jnp.float32)

    iota_sr = lax.broadcasted_iota(jnp.int32, (K0, N), 1)
    onehot = (out_idx_ref[...].reshape(K0, 1) == iota_sr).astype(jnp.float32)
    gt_ref[...] = lax.dot_general(w_out_ref[...], onehot,
                                  (((0,), (0,)), ((), ())),
                                  preferred_element_type=jnp.float32)


def _main(xt_ref, w0t_ref, b0_ref, bf_ref, ats_ref, ust_ref, gt_ref, bout_ref,
          outt_ref, st_t):
    f32 = jnp.float32
    tanht = jnp.tanh(jnp.dot(w0t_ref[...], xt_ref[...],
                             preferred_element_type=f32) + b0_ref[...])

    for q in range(N // Q):
        bq = pl.ds(q * Q, Q)
        acc = jnp.dot(ust_ref[bq, :], tanht,
                      preferred_element_type=f32) + bf_ref[bq, :]
        if q > 0:
            acc = acc + jnp.dot(ats_ref[bq, :q * Q], st_t[:q * Q, :],
                                preferred_element_type=f32)
        diag = ats_ref[bq, bq]
        v = acc
        for t in range(Q):
            r = jnp.maximum(v[t:t + 1, :], 0.0)
            v = v + diag[:, t:t + 1] * r
        st_t[bq, :] = jnp.maximum(v, 0.0)
    outt_ref[...] = jnp.dot(gt_ref[...], st_t[...],
                            preferred_element_type=f32) + bout_ref[...]


def kernel(X, W0, b0, W_hid, b_hid, W_out, b_out, in_idx, out_idx, write_pos):
    batch, input_dim = X.shape
    out_dim = W_out.shape[1]

    in_idxP = jnp.pad(in_idx, ((1, 1), (0, 0)))
    w_hidP = jnp.pad(W_hid, ((1, 1), (0, 0)))
    bfP = jnp.pad(b_hid, (1, 1)).reshape(N, 1)
    wp2 = write_pos.reshape(1, K0).astype(jnp.int32)
    oi2 = out_idx.reshape(1, K0).astype(jnp.int32)

    ats, ust, gt = pl.pallas_call(
        _builder,
        out_shape=(
            jax.ShapeDtypeStruct((N, N), jnp.float32),
            jax.ShapeDtypeStruct((N, K0), jnp.float32),
            jax.ShapeDtypeStruct((out_dim, N), jnp.float32),
        ),
    )(in_idxP, w_hidP, wp2, oi2, W_out)

    zero = lambda i: (0, 0)

    def run_main(xt_half):
        half = xt_half.shape[1]
        return pl.pallas_call(
            _main,
            grid=(half // BT,),
            in_specs=[
                pl.BlockSpec((input_dim, BT), lambda i: (0, i)),
                pl.BlockSpec((K0, input_dim), zero),
                pl.BlockSpec((K0, 1), zero),
                pl.BlockSpec((N, 1), zero),
                pl.BlockSpec((N, N), zero),
                pl.BlockSpec((N, K0), zero),
                pl.BlockSpec((out_dim, N), zero),
                pl.BlockSpec((out_dim, 1), zero),
            ],
            out_specs=(
                pl.BlockSpec((out_dim, BT), lambda i: (0, i)),
                pl.BlockSpec((N, BT), lambda i: (0, i)),
            ),
            out_shape=(
                jax.ShapeDtypeStruct((out_dim, half), jnp.float32),
                jax.ShapeDtypeStruct((N, half), jnp.float32),
            ),
        )(xt_half, W0.T, b0.reshape(K0, 1), bfP, ats, ust, gt,
          b_out.reshape(out_dim, 1))

    xt = X.T
    if batch % (2 * BT) == 0:
        h = batch // 2
        o1, s1 = run_main(xt[:, :h])
        o2, s2 = run_main(xt[:, h:])
        out = jnp.concatenate([o1.T, o2.T], axis=0)
        state = jnp.concatenate([s1.T, s2.T], axis=0)
        return (out, state)
    out_t, state_t = run_main(xt)
    return (out_t.T, state_t.T)


# speedup vs baseline: 1.3471x; 1.3471x over previous
import jax
import jax.numpy as jnp
from jax import lax
from jax.experimental import pallas as pl

N = 512
K = 32
K0 = 64
Q = 16
BT = 4096


def _builder(in_idx_ref, w_hid_ref, wp_ref, out_idx_ref, w_out_ref,
             ats_ref, ust_ref, gt_ref):
    iota_d = lax.broadcasted_iota(jnp.int32, (N, N), 0)
    iota_s = lax.broadcasted_iota(jnp.int32, (N, N), 1)
    at = jnp.zeros((N, N), jnp.float32)
    for k in range(K):
        idx_k = in_idx_ref[:, k:k + 1]
        at += w_hid_ref[:, k:k + 1] * (idx_k == iota_s).astype(jnp.float32)
    ats_ref[...] = at * (iota_s < iota_d).astype(jnp.float32)

    wp = wp_ref[...]
    iota_dc = lax.broadcasted_iota(jnp.int32, (N, K0), 0)
    ust = jnp.zeros((N, K0), jnp.float32)
    for k in range(K):
        idx_k = in_idx_ref[:, k:k + 1]
        ust += w_hid_ref[:, k:k + 1] * (idx_k == wp).astype(jnp.float32)
    ust_ref[...] = ust * (wp >= iota_dc).astype(jnp.float32)

    iota_sr = lax.broadcasted_iota(jnp.int32, (K0, N), 1)
    onehot = (out_idx_ref[...].reshape(K0, 1) == iota_sr).astype(jnp.float32)
    gt_ref[...] = lax.dot_general(w_out_ref[...], onehot,
                                  (((0,), (0,)), ((), ())),
                                  preferred_element_type=jnp.float32)


def _main(xt_ref, w0t_ref, b0_ref, bf_ref, ats_ref, ust_ref, gt_ref, bout_ref,
          outt_ref, st_t):
    f32 = jnp.float32
    tanht = jnp.tanh(jnp.dot(w0t_ref[...], xt_ref[...],
                             preferred_element_type=f32) + b0_ref[...])

    for q in range(N // Q):
        bq = pl.ds(q * Q, Q)
        acc = jnp.dot(ust_ref[bq, :], tanht,
                      preferred_element_type=f32) + bf_ref[bq, :]
        if q > 0:
            acc = acc + jnp.dot(ats_ref[bq, :q * Q], st_t[:q * Q, :],
                                preferred_element_type=f32)
        diag = ats_ref[bq, bq]
        v = acc
        for t in range(Q):
            r = jnp.maximum(v[t:t + 1, :], 0.0)
            v = v + diag[:, t:t + 1] * r
        st_t[bq, :] = jnp.maximum(v, 0.0)
    outt_ref[...] = jnp.dot(gt_ref[...], st_t[...],
                            preferred_element_type=f32) + bout_ref[...]


def kernel(X, W0, b0, W_hid, b_hid, W_out, b_out, in_idx, out_idx, write_pos):
    batch, input_dim = X.shape
    out_dim = W_out.shape[1]

    in_idxP = jnp.pad(in_idx, ((1, 1), (0, 0)))
    w_hidP = jnp.pad(W_hid, ((1, 1), (0, 0)))
    bfP = jnp.pad(b_hid, (1, 1)).reshape(N, 1)
    wp2 = write_pos.reshape(1, K0).astype(jnp.int32)
    oi2 = out_idx.reshape(1, K0).astype(jnp.int32)

    ats, ust, gt = pl.pallas_call(
        _builder,
        out_shape=(
            jax.ShapeDtypeStruct((N, N), jnp.float32),
            jax.ShapeDtypeStruct((N, K0), jnp.float32),
            jax.ShapeDtypeStruct((out_dim, N), jnp.float32),
        ),
    )(in_idxP, w_hidP, wp2, oi2, W_out)

    zero = lambda i: (0, 0)

    def run_main(xt_half):
        half = xt_half.shape[1]
        return pl.pallas_call(
            _main,
            grid=(half // BT,),
            in_specs=[
                pl.BlockSpec((input_dim, BT), lambda i: (0, i)),
                pl.BlockSpec((K0, input_dim), zero),
                pl.BlockSpec((K0, 1), zero),
                pl.BlockSpec((N, 1), zero),
                pl.BlockSpec((N, N), zero),
                pl.BlockSpec((N, K0), zero),
                pl.BlockSpec((out_dim, N), zero),
                pl.BlockSpec((out_dim, 1), zero),
            ],
            out_specs=(
                pl.BlockSpec((out_dim, BT), lambda i: (0, i)),
                pl.BlockSpec((N, BT), lambda i: (0, i)),
            ),
            out_shape=(
                jax.ShapeDtypeStruct((out_dim, half), jnp.float32),
                jax.ShapeDtypeStruct((N, half), jnp.float32),
            ),
        )(xt_half, W0.T, b0.reshape(K0, 1), bfP, ats, ust, gt,
          b_out.reshape(out_dim, 1))

    out_t, state_t = run_main(X.T)
    return (out_t.T, state_t.T)
